# Initial kernel scaffold; baseline (speedup 1.0000x reference)
import jax
import jax.numpy as jnp
from jax.experimental import pallas as pl


def kernel(x, edge_index, edge_weight, W1, b1, a1, W2, b2, a2, Wz, bz, Wr, br, Wh, bh, Wlin, blin):
    raise NotImplementedError("write your pallas kernel here")


# speedup vs baseline: 1.0221x; 1.0221x over previous
import functools

import jax
import jax.numpy as jnp
from jax.experimental import pallas as pl
from jax.experimental.pallas import tpu as pltpu

N = 100000
E = 3200000
F = 16
HID = 100
OC = 16

_NB = 5000


def _prelu(v, a):
    return jnp.where(v >= 0, v, a * v)


def _mlp_body(x_ref, w1_ref, b1_ref, a1_ref, w2_ref, b2_ref, a2_ref, o_ref):
    h = jnp.dot(x_ref[...], w1_ref[...], preferred_element_type=jnp.float32)
    h = _prelu(h + b1_ref[...][None, :], a1_ref[0])
    h = jnp.dot(h, w2_ref[...], preferred_element_type=jnp.float32)
    h = _prelu(h + b2_ref[...][None, :], a2_ref[0])
    o_ref[...] = h


def _mlp(x, W1, b1, a1, W2, b2, a2):
    grid = (N // _NB,)
    return pl.pallas_call(
        _mlp_body,
        grid=grid,
        in_specs=[
            pl.BlockSpec((_NB, F), lambda i: (i, 0)),
            pl.BlockSpec((F, HID), lambda i: (0, 0)),
            pl.BlockSpec((HID,), lambda i: (0,)),
            pl.BlockSpec((1,), lambda i: (0,)),
            pl.BlockSpec((HID, OC), lambda i: (0, 0)),
            pl.BlockSpec((OC,), lambda i: (0,)),
            pl.BlockSpec((1,), lambda i: (0,)),
        ],
        out_specs=pl.BlockSpec((_NB, OC), lambda i: (i, 0)),
        out_shape=jax.ShapeDtypeStruct((N, OC), jnp.float32),
    )(x, W1, b1, a1.reshape(1), W2, b2, a2.reshape(1))


def kernel(x, edge_index, edge_weight, W1, b1, a1, W2, b2, a2,
           Wz, bz, Wr, br, Wh, bh, Wlin, blin):
    row = edge_index[0]
    col = edge_index[1]

    h = _mlp(x, W1, b1, a1, W2, b2, a2)

    deg = jax.ops.segment_sum(edge_weight, row, num_segments=N)
    safe = jnp.where(deg > 0, deg, 1.0)
    dis = jnp.where(deg > 0, 1.0 / jnp.sqrt(safe), 0.0)
    A = dis[row] * edge_weight * dis[col]

    def prop(M):
        return jax.ops.segment_sum(A[:, None] * M[col], row, num_segments=N)

    def cheb3(X, PX, PPX, W, b, Hpart=0.0):
        out = (jnp.dot(X, W[0, :OC] - W[2, :OC])
               - jnp.dot(PX, W[1, :OC])
               + 2.0 * jnp.dot(PPX, W[2, :OC]) + b)
        return out + Hpart

    Ph = prop(h)
    PPh = prop(Ph)
    z1 = jax.nn.sigmoid(cheb3(h, Ph, PPh, Wz, bz))
    t1 = jnp.tanh(cheb3(h, Ph, PPh, Wh, bh))
    H1 = (1.0 - z1) * t1

    Px = prop(x)
    PPx = prop(Px)
    PH = prop(H1)
    PPH = prop(PH)

    def bot(PXh, PPXh, Xh, W):
        return (jnp.dot(Xh, W[0, OC:] - W[2, OC:])
                - jnp.dot(PXh, W[1, OC:])
                + 2.0 * jnp.dot(PPXh, W[2, OC:]))

    z2 = jax.nn.sigmoid(cheb3(x, Px, PPx, Wz, bz, bot(PH, PPH, H1, Wz)))
    r2 = jax.nn.sigmoid(cheb3(x, Px, PPx, Wr, br, bot(PH, PPH, H1, Wr)))
    Q = r2 * H1
    PQ = prop(Q)
    PPQ = prop(PQ)
    t2 = jnp.tanh(cheb3(x, Px, PPx, Wh, bh, bot(PQ, PPQ, Q, Wh)))
    H2 = z2 * H1 + (1.0 - z2) * t2

    out = jax.nn.relu(jnp.dot(H2, Wlin) + blin)
    return (out, A)


# speedup vs baseline: 61.8188x; 60.4796x over previous
import functools

import jax
import jax.numpy as jnp
from jax import lax
from jax.experimental import pallas as pl
from jax.experimental.pallas import tpu as pltpu
from jax.experimental.pallas import tpu_sc as plsc

N = 100000
E = 3200000
F = 16
HID = 100
OC = 16

NC = 2
NS = 16
NW = NC * NS
EC = 128
MC = 8

_NB = 2000

_mesh = plsc.VectorSubcoreMesh(core_axis_name="c", subcore_axis_name="s",
                               num_cores=NC)

_SC_PARAMS = pltpu.CompilerParams(use_tc_tiling_on_sc=False)
_SC_PARAMS_NL = pltpu.CompilerParams(use_tc_tiling_on_sc=False,
                                     needs_layout_passes=False)


def _pad_edges(row, col, ew, n_edges):
    nchunk = n_edges // EC
    n_macros = -(-nchunk // (NW * MC))
    npad = n_macros * NW * MC
    epad = npad * EC
    pr = jnp.pad(row, (0, epad - n_edges)).reshape(npad, EC)
    pc = jnp.pad(col, (0, epad - n_edges)).reshape(npad, EC)
    pack = jnp.stack([pr, pc], axis=1)
    ew2d = jnp.pad(ew, (0, epad - n_edges)).reshape(npad, EC)
    return pack, ew2d, n_macros


def _make_prop(n_nodes, n_macros):
    cpw = n_macros * MC
    rps = n_nodes // NS
    zr = max(d for d in range(1, rps + 1)
             if rps % d == 0 and d * OC * 4 <= 8192)
    ncopy = rps // zr
    assert rps * NS == n_nodes and zr * ncopy == rps

    @functools.partial(
        pl.kernel,
        out_type=jax.ShapeDtypeStruct((NC, n_nodes, OC), jnp.float32),
        mesh=_mesh,
        compiler_params=_SC_PARAMS,
        scratch_types=[
            pltpu.VMEM((2, MC, 2, EC), jnp.int32),
            pltpu.VMEM((2, MC, EC), jnp.float32),
            pltpu.VMEM((MC, EC, OC), jnp.float32),
            pltpu.VMEM((zr, OC), jnp.float32),
            pltpu.VMEM_SHARED((n_nodes, OC), jnp.float32),
            pltpu.SemaphoreType.DMA,
            pltpu.SemaphoreType.DMA,
            pltpu.SemaphoreType.DMA,
        ],
    )
    def prop(m_hbm, pack_hbm, a2d_hbm, out_hbm,
             pbuf, abuf, rows, zbuf, acc, sem_p, sem_g, sem_s):
        c = lax.axis_index("c")
        s = lax.axis_index("s")
        w = _worker(c, s)
        base = w * cpw

        pltpu.async_copy(pack_hbm.at[pl.ds(base, MC)], pbuf.at[0], sem_p)
        pltpu.async_copy(a2d_hbm.at[pl.ds(base, MC)], abuf.at[0], sem_p)

        @plsc.parallel_loop(0, zr, unroll=8)
        def _zinit(i):
            zbuf[i, :] = jnp.zeros((OC,), jnp.float32)

        zdescs = [pltpu.async_copy(zbuf, acc.at[pl.ds(s * rps + j * zr, zr)],
                                   sem_g) for j in range(ncopy)]
        for d in zdescs:
            d.wait()
        plsc.subcore_barrier()

        def macro(m, carry):
            b = m & 1
            pltpu.make_async_copy(pack_hbm.at[pl.ds(base + m * MC, MC)],
                                  pbuf.at[b], sem_p).wait()
            pltpu.make_async_copy(a2d_hbm.at[pl.ds(base + m * MC, MC)],
                                  abuf.at[b], sem_p).wait()

            descs = []
            for j in range(MC):
                @pl.when(m > 0)
                def _():
                    pltpu.make_async_copy(m_hbm.at[pl.ds(0, EC)],
                                          rows.at[j], sem_s).wait()
                descs.append(pltpu.async_copy(
                    m_hbm.at[pbuf.at[b, j, 1]], rows.at[j], sem_g))

            @pl.when(m < n_macros - 1)
            def _():
                nb = (m + 1) & 1
                pltpu.async_copy(pack_hbm.at[pl.ds(base + (m + 1) * MC, MC)],
                                 pbuf.at[nb], sem_p)
                pltpu.async_copy(a2d_hbm.at[pl.ds(base + (m + 1) * MC, MC)],
                                 abuf.at[nb], sem_p)
            for j in range(MC):
                descs[j].wait()

                @plsc.parallel_loop(0, EC // 16, unroll=4)
                def _scale(g):
                    avec = abuf[b, j, pl.ds(g * 16, 16)]
                    for k in range(16):
                        e = g * 16 + k
                        rows[j, e, :] = rows[j, e, :] * avec[k]

                pltpu.async_copy(rows.at[j], acc.at[pbuf.at[b, j, 0]],
                                 sem_s, add=True)
            return carry

        lax.fori_loop(0, n_macros, macro, 0)
        for j in range(MC):
            pltpu.make_async_copy(m_hbm.at[pl.ds(0, EC)],
                                  rows.at[j], sem_s).wait()
        plsc.subcore_barrier()
        pltpu.sync_copy(acc.at[pl.ds(s * rps, rps)],
                        out_hbm.at[c, pl.ds(s * rps, rps)])

    return prop


def _make_deg(n_nodes, n_macros):
    cpw = n_macros * MC
    g = n_nodes // 8
    base_g = g // NS
    extra = g - base_g * NS
    big = (base_g + 1) * 8
    small = base_g * 8

    @functools.partial(
        pl.kernel,
        out_type=jax.ShapeDtypeStruct((NC, n_nodes), jnp.float32),
        mesh=_mesh,
        compiler_params=_SC_PARAMS,
        scratch_types=[
            pltpu.VMEM((2, MC, 2, EC), jnp.int32),
            pltpu.VMEM((2, MC, EC), jnp.float32),
            pltpu.VMEM((big,), jnp.float32),
            pltpu.VMEM_SHARED((n_nodes,), jnp.float32),
            pltpu.SemaphoreType.DMA,
            pltpu.SemaphoreType.DMA,
        ],
    )
    def deg(pack_hbm, ew_hbm, out_hbm, pbuf, ebuf, zbuf, acc, sem_p, sem_s):
        c = lax.axis_index("c")
        s = lax.axis_index("s")
        w = _worker(c, s)
        base = w * cpw

        @plsc.parallel_loop(0, big // 16, unroll=8)
        def _zinit(i):
            zbuf[pl.ds(i * 16, 16)] = jnp.zeros((16,), jnp.float32)

        off = jnp.where(s < extra, s * big, extra * big + (s - extra) * small)

        @pl.when(s < extra)
        def _():
            pltpu.sync_copy(zbuf, acc.at[pl.ds(off, big)])

        @pl.when(s >= extra)
        def _():
            pltpu.sync_copy(zbuf.at[pl.ds(0, small)], acc.at[pl.ds(off, small)])

        plsc.subcore_barrier()

        pltpu.async_copy(pack_hbm.at[pl.ds(base, MC)], pbuf.at[0], sem_p)
        pltpu.async_copy(ew_hbm.at[pl.ds(base, MC)], ebuf.at[0], sem_p)

        def macro(m, carry):
            b = m & 1
            pltpu.make_async_copy(pack_hbm.at[pl.ds(base + m * MC, MC)],
                                  pbuf.at[b], sem_p).wait()
            pltpu.make_async_copy(ew_hbm.at[pl.ds(base + m * MC, MC)],
                                  ebuf.at[b], sem_p).wait()

            @pl.when(m > 0)
            def _():
                for j in range(MC):
                    pltpu.make_async_copy(ew_hbm.at[0], ebuf.at[0, j],
                                          sem_s).wait()

            @pl.when(m < n_macros - 1)
            def _():
                nb = (m + 1) & 1
                pltpu.async_copy(pack_hbm.at[pl.ds(base + (m + 1) * MC, MC)],
                                 pbuf.at[nb], sem_p)
                pltpu.async_copy(ew_hbm.at[pl.ds(base + (m + 1) * MC, MC)],
                                 ebuf.at[nb], sem_p)

            for j in range(MC):
                pltpu.async_copy(ebuf.at[b, j], acc.at[pbuf.at[b, j, 0]],
                                 sem_s, add=True)
            return carry

        lax.fori_loop(0, n_macros, macro, 0)
        for j in range(MC):
            pltpu.make_async_copy(ew_hbm.at[0], ebuf.at[0, j], sem_s).wait()
        plsc.subcore_barrier()

        @pl.when(s < extra)
        def _():
            pltpu.sync_copy(acc.at[pl.ds(off, big)],
                            out_hbm.at[c, pl.ds(off, big)])

        @pl.when(s >= extra)
        def _():
            pltpu.sync_copy(acc.at[pl.ds(off, small)],
                            out_hbm.at[c, pl.ds(off, small)])

    return deg


def _make_edge_norm(n_nodes, n_macros, npad):
    cpw = n_macros * MC

    @functools.partial(
        pl.kernel,
        out_type=jax.ShapeDtypeStruct((npad, EC), jnp.float32),
        mesh=_mesh,
        compiler_params=_SC_PARAMS_NL,
        scratch_types=[
            pltpu.VMEM((2, MC, 2, EC), jnp.int32),
            pltpu.VMEM((2, MC, EC), jnp.float32),
            pltpu.VMEM((MC, EC), jnp.float32),
            pltpu.VMEM((n_nodes,), jnp.float32),
            pltpu.SemaphoreType.DMA,
        ],
    )
    def edge_norm(dis_hbm, pack_hbm, ew_hbm, a_hbm,
                  pbuf, ebuf, obuf, disv, sem_p):
        c = lax.axis_index("c")
        s = lax.axis_index("s")
        w = _worker(c, s)
        base = w * cpw
        pltpu.sync_copy(dis_hbm, disv)

        pltpu.async_copy(pack_hbm.at[pl.ds(base, MC)], pbuf.at[0], sem_p)
        pltpu.async_copy(ew_hbm.at[pl.ds(base, MC)], ebuf.at[0], sem_p)

        def macro(m, carry):
            b = m & 1
            pltpu.make_async_copy(pack_hbm.at[pl.ds(base + m * MC, MC)],
                                  pbuf.at[b], sem_p).wait()
            pltpu.make_async_copy(ew_hbm.at[pl.ds(base + m * MC, MC)],
                                  ebuf.at[b], sem_p).wait()

            @pl.when(m < n_macros - 1)
            def _():
                nb = (m + 1) & 1
                pltpu.async_copy(pack_hbm.at[pl.ds(base + (m + 1) * MC, MC)],
                                 pbuf.at[nb], sem_p)
                pltpu.async_copy(ew_hbm.at[pl.ds(base + (m + 1) * MC, MC)],
                                 ebuf.at[nb], sem_p)

            for j in range(MC):
                for gi in range(EC // 16):
                    sl = pl.ds(gi * 16, 16)
                    dr = plsc.load_gather(disv, [pbuf[b, j, 0, sl]])
                    dc = plsc.load_gather(disv, [pbuf[b, j, 1, sl]])
                    obuf[j, sl] = dr * ebuf[b, j, sl] * dc
            pltpu.sync_copy(obuf, a_hbm.at[pl.ds(base + m * MC, MC)])
            return carry

        lax.fori_loop(0, n_macros, macro, 0)

    return edge_norm


def _worker(c, s):
    return s * NC + c


_N_MACROS = -(-(E // EC) // (NW * MC))
_NPAD = _N_MACROS * NW * MC
_prop_sc = _make_prop(N, _N_MACROS)
_deg_sc = _make_deg(N, _N_MACROS)
_edge_norm_sc = _make_edge_norm(N, _N_MACROS, _NPAD)


def _prelu(v, a):
    return jnp.where(v >= 0, v, a * v)


def _mlp_body(x_ref, w1_ref, b1_ref, a1_ref, w2_ref, b2_ref, a2_ref, o_ref):
    h = jnp.dot(x_ref[...], w1_ref[...], preferred_element_type=jnp.float32)
    h = _prelu(h + b1_ref[...][None, :], a1_ref[0])
    h = jnp.dot(h, w2_ref[...], preferred_element_type=jnp.float32)
    h = _prelu(h + b2_ref[...][None, :], a2_ref[0])
    o_ref[...] = h


def _mlp(x, W1, b1, a1, W2, b2, a2):
    return pl.pallas_call(
        _mlp_body,
        grid=(N // _NB,),
        in_specs=[
            pl.BlockSpec((_NB, F), lambda i: (i, 0)),
            pl.BlockSpec((F, HID), lambda i: (0, 0)),
            pl.BlockSpec((HID,), lambda i: (0,)),
            pl.BlockSpec((1,), lambda i: (0,)),
            pl.BlockSpec((HID, OC), lambda i: (0, 0)),
            pl.BlockSpec((OC,), lambda i: (0,)),
            pl.BlockSpec((1,), lambda i: (0,)),
        ],
        out_specs=pl.BlockSpec((_NB, OC), lambda i: (i, 0)),
        out_shape=jax.ShapeDtypeStruct((N, OC), jnp.float32),
    )(x, W1, b1, a1.reshape(1), W2, b2, a2.reshape(1))


def _dis_body(degp_ref, o_ref):
    deg = degp_ref[0, :] + degp_ref[1, :]
    safe = jnp.where(deg > 0, deg, 1.0)
    o_ref[...] = jnp.where(deg > 0, lax.rsqrt(safe), 0.0)


def _dis(deg_parts):
    return pl.pallas_call(
        _dis_body,
        out_shape=jax.ShapeDtypeStruct((N,), jnp.float32),
    )(deg_parts)


def _combine_body(p_ref, o_ref):
    o_ref[...] = p_ref[0] + p_ref[1]


def _combine(parts):
    return pl.pallas_call(
        _combine_body,
        grid=(N // _NB,),
        in_specs=[pl.BlockSpec((NC, _NB, OC), lambda i: (0, i, 0))],
        out_specs=pl.BlockSpec((_NB, OC), lambda i: (i, 0)),
        out_shape=jax.ShapeDtypeStruct((N, OC), jnp.float32),
    )(parts)


def _top(Wk):
    return Wk[:OC]


def _bot(Wk):
    return Wk[OC:]


def _cheb_top(Xb, PXb, PPXb, W, b):
    return (jnp.dot(Xb, _top(W[0]) - _top(W[2]), preferred_element_type=jnp.float32)
            - jnp.dot(PXb, _top(W[1]), preferred_element_type=jnp.float32)
            + 2.0 * jnp.dot(PPXb, _top(W[2]), preferred_element_type=jnp.float32)
            + b[None, :])


def _cheb_bot(Xb, PXb, PPXb, W):
    return (jnp.dot(Xb, _bot(W[0]) - _bot(W[2]), preferred_element_type=jnp.float32)
            - jnp.dot(PXb, _bot(W[1]), preferred_element_type=jnp.float32)
            + 2.0 * jnp.dot(PPXb, _bot(W[2]), preferred_element_type=jnp.float32))


def _cell1_body(h_ref, ph_ref, pph_ref, wz_ref, bz_ref, wh_ref, bh_ref, o_ref):
    hb = h_ref[...]
    phb = ph_ref[...]
    pphb = pph_ref[0] + pph_ref[1]
    z1 = jax.nn.sigmoid(_cheb_top(hb, phb, pphb, wz_ref, bz_ref[...]))
    t1 = jnp.tanh(_cheb_top(hb, phb, pphb, wh_ref, bh_ref[...]))
    o_ref[...] = (1.0 - z1) * t1


def _cell1(h, Phs, PPh, Wz, bz, Wh, bh):
    wspec = pl.BlockSpec((3, 2 * OC, OC), lambda i: (0, 0, 0))
    bspec = pl.BlockSpec((OC,), lambda i: (0,))
    nspec = pl.BlockSpec((_NB, OC), lambda i: (i, 0))
    pspec = pl.BlockSpec((NC, _NB, OC), lambda i: (0, i, 0))
    return pl.pallas_call(
        _cell1_body,
        grid=(N // _NB,),
        in_specs=[nspec, nspec, pspec, wspec, bspec, wspec, bspec],
        out_specs=nspec,
        out_shape=jax.ShapeDtypeStruct((N, OC), jnp.float32),
    )(h, Phs, PPh, Wz, bz, Wh, bh)


def _gates2_body(x_ref, h1_ref, px_ref, ppx_ref, ph_ref, pph_ref,
                 wz_ref, bz_ref, wr_ref, br_ref, wh_ref, bh_ref,
                 z2_ref, q_ref, tpart_ref):
    xb = x_ref[...]
    h1 = h1_ref[...]
    pxb = px_ref[...]
    ppxb = ppx_ref[0] + ppx_ref[1]
    phb = ph_ref[...]
    pphb = pph_ref[0] + pph_ref[1]
    z2 = jax.nn.sigmoid(_cheb_top(xb, pxb, ppxb, wz_ref, bz_ref[...])
                        + _cheb_bot(h1, phb, pphb, wz_ref))
    r2 = jax.nn.sigmoid(_cheb_top(xb, pxb, ppxb, wr_ref, br_ref[...])
                        + _cheb_bot(h1, phb, pphb, wr_ref))
    z2_ref[...] = z2
    q_ref[...] = r2 * h1
    tpart_ref[...] = _cheb_top(xb, pxb, ppxb, wh_ref, bh_ref[...])


def _gates2(x, H1, Pxs, PPx, PHs, PPH, Wz, bz, Wr, br, Wh, bh):
    wspec = pl.BlockSpec((3, 2 * OC, OC), lambda i: (0, 0, 0))
    bspec = pl.BlockSpec((OC,), lambda i: (0,))
    nspec = pl.BlockSpec((_NB, OC), lambda i: (i, 0))
    pspec = pl.BlockSpec((NC, _NB, OC), lambda i: (0, i, 0))
    sh = jax.ShapeDtypeStruct((N, OC), jnp.float32)
    return pl.pallas_call(
        _gates2_body,
        grid=(N // _NB,),
        in_specs=[nspec, nspec, nspec, pspec, nspec, pspec,
                  wspec, bspec, wspec, bspec, wspec, bspec],
        out_specs=(nspec, nspec, nspec),
        out_shape=(sh, sh, sh),
    )(x, H1, Pxs, PPx, PHs, PPH, Wz, bz, Wr, br, Wh, bh)


def _final_body(tpart_ref, z2_ref, h1_ref, q_ref, pq_ref, ppq_ref,
                wh_ref, wlin_ref, blin_ref, o_ref):
    qb = q_ref[...]
    pqb = pq_ref[...]
    ppqb = ppq_ref[0] + ppq_ref[1]
    t2 = jnp.tanh(tpart_ref[...] + _cheb_bot(qb, pqb, ppqb, wh_ref))
    z2 = z2_ref[...]
    h2 = z2 * h1_ref[...] + (1.0 - z2) * t2
    o_ref[...] = jnp.maximum(
        jnp.dot(h2, wlin_ref[...], preferred_element_type=jnp.float32)
        + blin_ref[...][None, :], 0.0)


def _final(tpart, z2, H1, Q, PQs, PPQ, Wh, Wlin, blin):
    wspec = pl.BlockSpec((3, 2 * OC, OC), lambda i: (0, 0, 0))
    nspec = pl.BlockSpec((_NB, OC), lambda i: (i, 0))
    pspec = pl.BlockSpec((NC, _NB, OC), lambda i: (0, i, 0))
    return pl.pallas_call(
        _final_body,
        grid=(N // _NB,),
        in_specs=[nspec, nspec, nspec, nspec, nspec, pspec, wspec,
                  pl.BlockSpec((OC, 1), lambda i: (0, 0)),
                  pl.BlockSpec((1,), lambda i: (0,))],
        out_specs=pl.BlockSpec((_NB, 1), lambda i: (i, 0)),
        out_shape=jax.ShapeDtypeStruct((N, 1), jnp.float32),
    )(tpart, z2, H1, Q, PQs, PPQ, Wh, Wlin, blin)


def kernel(x, edge_index, edge_weight, W1, b1, a1, W2, b2, a2,
           Wz, bz, Wr, br, Wh, bh, Wlin, blin):
    row = edge_index[0]
    col = edge_index[1]
    pack, ew2d, n_macros = _pad_edges(row, col, edge_weight, E)
    assert n_macros == _N_MACROS

    h = _mlp(x, W1, b1, a1, W2, b2, a2)

    deg_parts = _deg_sc(pack, ew2d)
    dis = _dis(deg_parts)
    a2d = _edge_norm_sc(dis, pack, ew2d)
    A = a2d.reshape(-1)[:E]

    def prop2(M):
        Pm = _combine(_prop_sc(M, pack, a2d))
        PPm = _prop_sc(Pm, pack, a2d)
        return Pm, PPm

    Phs, PPh = prop2(h)
    H1 = _cell1(h, Phs, PPh, Wz, bz, Wh, bh)

    Pxs, PPx = prop2(x)
    PHs, PPH = prop2(H1)
    z2, Q, tpart = _gates2(x, H1, Pxs, PPx, PHs, PPH, Wz, bz, Wr, br, Wh, bh)

    PQs, PPQ = prop2(Q)
    out = _final(tpart, z2, H1, Q, PQs, PPQ, Wh, Wlin, blin)
    return (out, A)
